# Initial kernel scaffold; baseline (speedup 1.0000x reference)
import jax
import jax.numpy as jnp
from jax.experimental import pallas as pl


def kernel(x, table):
    raise NotImplementedError("write your pallas kernel here")


# speedup vs baseline: 3.5427x; 3.5427x over previous
import functools

import jax
import jax.numpy as jnp
from jax import lax
from jax.experimental import pallas as pl
from jax.experimental.pallas import tpu as pltpu
from jax.experimental.pallas import tpu_sc as plsc

VOCAB = 100000
D = 64
B = 4096 * 200
NC = 2
NS = 16
NW = NC * NS
BPW = B // NW
CHUNK = 128
NCHUNK = BPW // CHUNK

_mesh = plsc.VectorSubcoreMesh(core_axis_name="c", subcore_axis_name="s")


@functools.partial(
    pl.kernel,
    mesh=_mesh,
    out_type=jax.ShapeDtypeStruct((B, D), jnp.float32),
    scratch_types=[
        pltpu.VMEM((BPW,), jnp.int32),
        pltpu.VMEM((CHUNK, D), jnp.float32),
        pltpu.SemaphoreType.DMA,
    ],
    compiler_params=pltpu.CompilerParams(use_tc_tiling_on_sc=False),
)
def _gather_kernel(idx_hbm, table_hbm, out_hbm, idx_v, rows_v, sem):
    wid = lax.axis_index("s") * NC + lax.axis_index("c")
    base = wid * BPW
    pltpu.sync_copy(idx_hbm.at[pl.ds(base, BPW)], idx_v)

    def body(j, carry):
        idx_slice = idx_v.at[pl.ds(j * CHUNK, CHUNK)]
        pltpu.async_copy(table_hbm.at[idx_slice], rows_v, sem).wait()
        pltpu.sync_copy(rows_v, out_hbm.at[pl.ds(base + j * CHUNK, CHUNK)])
        return carry

    lax.fori_loop(0, NCHUNK, body, 0)


def kernel(x, table):
    flat = x.reshape(-1)
    out = _gather_kernel(flat, table)
    return out.reshape(x.shape + (table.shape[1],))


# speedup vs baseline: 4.1913x; 1.1831x over previous
import functools

import jax
import jax.numpy as jnp
from jax import lax
from jax.experimental import pallas as pl
from jax.experimental.pallas import tpu as pltpu
from jax.experimental.pallas import tpu_sc as plsc

VOCAB = 100000
D = 64
B = 4096 * 200
NC = 2
NS = 16
NW = NC * NS
BPW = B // NW
CHUNK = 128
NCHUNK = BPW // CHUNK
NBUF = 8
NGROUP = NCHUNK // NBUF

_mesh = plsc.VectorSubcoreMesh(core_axis_name="c", subcore_axis_name="s")


@functools.partial(
    pl.kernel,
    mesh=_mesh,
    out_type=jax.ShapeDtypeStruct((B, D), jnp.float32),
    scratch_types=[
        pltpu.VMEM((BPW,), jnp.int32),
        pltpu.VMEM((NBUF * CHUNK, D), jnp.float32),
        pltpu.SemaphoreType.DMA,
        pltpu.SemaphoreType.DMA,
    ],
    compiler_params=pltpu.CompilerParams(use_tc_tiling_on_sc=False),
)
def _gather_kernel(idx_hbm, table_hbm, out_hbm, idx_v, rows_v, sem_g, sem_s):
    wid = lax.axis_index("s") * NC + lax.axis_index("c")
    base = wid * BPW
    pltpu.sync_copy(idx_hbm.at[pl.ds(base, BPW)], idx_v)

    def group(g, carry):
        cbase = g * (NBUF * CHUNK)

        @pl.when(g > 0)
        def _drain_prev_stores():
            for b in range(NBUF):
                pltpu.make_async_copy(
                    rows_v.at[pl.ds(b * CHUNK, CHUNK)],
                    out_hbm.at[pl.ds(base, CHUNK)],
                    sem_s,
                ).wait()

        copies = []
        for b in range(NBUF):
            idx_slice = idx_v.at[pl.ds(cbase + b * CHUNK, CHUNK)]
            copies.append(
                pltpu.async_copy(
                    table_hbm.at[idx_slice],
                    rows_v.at[pl.ds(b * CHUNK, CHUNK)],
                    sem_g,
                )
            )
        for c in copies:
            c.wait()
        for b in range(NBUF):
            pltpu.async_copy(
                rows_v.at[pl.ds(b * CHUNK, CHUNK)],
                out_hbm.at[pl.ds(base + cbase + b * CHUNK, CHUNK)],
                sem_s,
            )
        return carry

    lax.fori_loop(0, NGROUP, group, 0)

    for b in range(NBUF):
        pltpu.make_async_copy(
            rows_v.at[pl.ds(b * CHUNK, CHUNK)],
            out_hbm.at[pl.ds(base, CHUNK)],
            sem_s,
        ).wait()


def kernel(x, table):
    flat = x.reshape(-1)
    out = _gather_kernel(flat, table)
    return out.reshape(x.shape + (table.shape[1],))


# speedup vs baseline: 4.2247x; 1.0080x over previous
import functools

import jax
import jax.numpy as jnp
from jax import lax
from jax.experimental import pallas as pl
from jax.experimental.pallas import tpu as pltpu
from jax.experimental.pallas import tpu_sc as plsc

VOCAB = 100000
D = 64
B = 4096 * 200
NC = 2
NS = 16
NW = NC * NS
BPW = B // NW
CHUNK = 128
NCHUNK = BPW // CHUNK
NBUF = 10
NGROUP = NCHUNK // NBUF

_mesh = plsc.VectorSubcoreMesh(core_axis_name="c", subcore_axis_name="s")


@functools.partial(
    pl.kernel,
    mesh=_mesh,
    out_type=jax.ShapeDtypeStruct((B, D), jnp.float32),
    scratch_types=[
        pltpu.VMEM((BPW,), jnp.int32),
        pltpu.VMEM((NBUF * CHUNK, D), jnp.float32),
        pltpu.SemaphoreType.DMA,
        pltpu.SemaphoreType.DMA,
    ],
    compiler_params=pltpu.CompilerParams(use_tc_tiling_on_sc=False),
)
def _gather_kernel(idx_hbm, table_hbm, out_hbm, idx_v, rows_v, sem_g, sem_s):
    wid = lax.axis_index("s") * NC + lax.axis_index("c")
    base = wid * BPW
    pltpu.sync_copy(idx_hbm.at[pl.ds(base, BPW)], idx_v)

    def group(g, carry):
        cbase = g * (NBUF * CHUNK)

        @pl.when(g > 0)
        def _drain_prev_stores():
            for b in range(NBUF):
                pltpu.make_async_copy(
                    rows_v.at[pl.ds(b * CHUNK, CHUNK)],
                    out_hbm.at[pl.ds(base, CHUNK)],
                    sem_s,
                ).wait()

        copies = []
        for b in range(NBUF):
            idx_slice = idx_v.at[pl.ds(cbase + b * CHUNK, CHUNK)]
            copies.append(
                pltpu.async_copy(
                    table_hbm.at[idx_slice],
                    rows_v.at[pl.ds(b * CHUNK, CHUNK)],
                    sem_g,
                )
            )
        for c in copies:
            c.wait()
        for b in range(NBUF):
            pltpu.async_copy(
                rows_v.at[pl.ds(b * CHUNK, CHUNK)],
                out_hbm.at[pl.ds(base + cbase + b * CHUNK, CHUNK)],
                sem_s,
            )
        return carry

    lax.fori_loop(0, NGROUP, group, 0)

    for b in range(NBUF):
        pltpu.make_async_copy(
            rows_v.at[pl.ds(b * CHUNK, CHUNK)],
            out_hbm.at[pl.ds(base, CHUNK)],
            sem_s,
        ).wait()


def kernel(x, table):
    flat = x.reshape(-1)
    out = _gather_kernel(flat, table)
    return out.reshape(x.shape + (table.shape[1],))


# speedup vs baseline: 4.4122x; 1.0444x over previous
import functools

import jax
import jax.numpy as jnp
from jax import lax
from jax.experimental import pallas as pl
from jax.experimental.pallas import tpu as pltpu
from jax.experimental.pallas import tpu_sc as plsc

VOCAB = 100000
D = 64
B = 4096 * 200
NC = 2
NS = 16
NW = NC * NS
BPW = B // NW
CHUNK = 128
NCHUNK = BPW // CHUNK
NBUF = 10
NGROUP = NCHUNK // NBUF

_mesh = plsc.VectorSubcoreMesh(core_axis_name="c", subcore_axis_name="s")


@functools.partial(
    pl.kernel,
    mesh=_mesh,
    out_type=jax.ShapeDtypeStruct((B, D), jnp.float32),
    scratch_types=[
        pltpu.VMEM((BPW,), jnp.int32),
        pltpu.VMEM((NBUF * CHUNK, D), jnp.float32),
        pltpu.SemaphoreType.DMA,
        pltpu.SemaphoreType.DMA,
    ],
    compiler_params=pltpu.CompilerParams(use_tc_tiling_on_sc=False),
)
def _gather_kernel(idx_hbm, table_hbm, out_hbm, idx_v, rows_v, sem_g, sem_s):
    wid = lax.axis_index("s") * NC + lax.axis_index("c")
    base = wid * BPW
    pltpu.sync_copy(idx_hbm.at[pl.ds(base, BPW)], idx_v)

    def group(g, carry):
        cbase = g * (NBUF * CHUNK)

        @pl.when(g > 0)
        def _drain_prev_stores():
            for b in range(NBUF):
                pltpu.make_async_copy(
                    rows_v.at[pl.ds(b * CHUNK, CHUNK)],
                    out_hbm.at[pl.ds(base, CHUNK)],
                    sem_s,
                ).wait()

        copies = []
        for b in range(NBUF):
            idx_slice = idx_v.at[pl.ds(cbase + b * CHUNK, CHUNK)]
            copies.append(
                pltpu.async_copy(
                    table_hbm.at[idx_slice],
                    rows_v.at[pl.ds(b * CHUNK, CHUNK)],
                    sem_g,
                )
            )
        for c in copies:
            c.wait()
        for b in range(NBUF):
            pltpu.async_copy(
                rows_v.at[pl.ds(b * CHUNK, CHUNK)],
                out_hbm.at[pl.ds(base + cbase + b * CHUNK, CHUNK)],
                sem_s,
            )
        return carry

    lax.fori_loop(0, NGROUP, group, 0)

    for b in range(NBUF):
        pltpu.make_async_copy(
            rows_v.at[pl.ds(b * CHUNK, CHUNK)],
            out_hbm.at[pl.ds(base, CHUNK)],
            sem_s,
        ).wait()


def kernel(x, table):
    flat = x.T.reshape(-1)
    out = _gather_kernel(flat, table)
    seq, batch = x.shape[1], x.shape[0]
    return out.reshape(seq, batch, table.shape[1]).transpose(1, 0, 2)
